# baseline (speedup 1.0000x reference)
import functools

import jax
import jax.numpy as jnp
from jax import lax
from jax.experimental import pallas as pl
from jax.experimental.pallas import tpu as pltpu
from jax.experimental.pallas import tpu_sc as plsc


def _sc_embed(ids_flat, token_table, position_table, *, B, L, D):
    NC, NS = 2, 16
    NW = NC * NS
    BPW = B // NW
    NP = BPW // 2
    n_rows = BPW * L

    mesh = plsc.VectorSubcoreMesh(core_axis_name="c", subcore_axis_name="s")

    @functools.partial(
        pl.kernel,
        mesh=mesh,
        out_type=jax.ShapeDtypeStruct((B * L, D), jnp.float32),
        scratch_types=[
            pltpu.VMEM((2 * L,), jnp.int32),
            pltpu.VMEM((2 * L,), jnp.int32),
            pltpu.VMEM((L, D), jnp.float32),
        ] + [pltpu.VMEM((L, D), jnp.float32) for _ in range(4)]
          + [pltpu.SemaphoreType.DMA for _ in range(4 + 4 + 2 + 1)],
    )
    def k(ids_hbm, tbl_hbm, pos_hbm, out_hbm, idx0, idx1, pos_v, *rest):
        bufs = rest[:4]
        gsem = rest[4:8]
        ssem = rest[8:12]
        isem = rest[12:14]
        psem = rest[14]
        idxs = (idx0, idx1)

        wid = lax.axis_index("s") * NC + lax.axis_index("c")
        base = wid * n_rows

        pos_cp = pltpu.async_copy(pos_hbm.at[pl.ds(0, L)], pos_v, psem)

        def load_idx(p):
            return pltpu.async_copy(
                ids_hbm.at[pl.ds(base + 2 * p * L, 2 * L)],
                idxs[p % 2], isem[p % 2])

        def issue_pair_gathers(p):
            ib = idxs[p % 2]
            cps = []
            for half in range(2):
                bslot = 2 * (p % 2) + half
                buf = bufs[bslot]
                off = half * L
                cps.append(pltpu.async_copy(
                    tbl_hbm.at[ib.at[pl.ds(off, 128)]],
                    buf.at[pl.ds(0, 128)], gsem[bslot]))
                cps.append(pltpu.async_copy(
                    tbl_hbm.at[ib.at[pl.ds(off + 128, L - 128)]],
                    buf.at[pl.ds(128, L - 128)], gsem[bslot]))
            return cps

        icp = load_idx(0)
        icp.wait()
        gathers = {0: issue_pair_gathers(0)}
        idx_cps = {1: load_idx(1)}
        scatters = {}
        pos_cp.wait()

        for p in range(NP):
            for cp in gathers.pop(p):
                cp.wait()

            B0 = bufs[2 * (p % 2)]
            B1 = bufs[2 * (p % 2) + 1]

            def add_row(l, carry, B0=B0, B1=B1):
                for j in range(D // 16):
                    sl = pl.ds(j * 16, 16)
                    pv = pos_v[l, sl]
                    B0[l, sl] = B0[l, sl] + pv
                    B1[l, sl] = B1[l, sl] + pv
                return carry

            lax.fori_loop(0, L, add_row, 0)

            scatters[p] = (
                pltpu.async_copy(
                    B0, out_hbm.at[pl.ds(base + 2 * p * L, L)],
                    ssem[2 * (p % 2)]),
                pltpu.async_copy(
                    B1, out_hbm.at[pl.ds(base + (2 * p + 1) * L, L)],
                    ssem[2 * (p % 2) + 1]),
            )

            if p + 1 < NP:
                if p >= 1:
                    for cp in scatters.pop(p - 1):
                        cp.wait()
                idx_cps.pop(p + 1).wait()
                gathers[p + 1] = issue_pair_gathers(p + 1)
                if p + 2 < NP:
                    idx_cps[p + 2] = load_idx(p + 2)

        for p in sorted(scatters):
            for cp in scatters[p]:
                cp.wait()

    return k(ids_flat, token_table, position_table)


def kernel(input_ids, token_table, position_table):
    B, L = input_ids.shape
    _, D = token_table.shape
    ids_flat = input_ids.reshape(B * L).astype(jnp.int32)
    out = _sc_embed(ids_flat, token_table.astype(jnp.float32),
                    position_table.astype(jnp.float32), B=B, L=L, D=D)
    return out.reshape(B, L, D)


# speedup vs baseline: 1.2027x; 1.2027x over previous
import functools

import jax
import jax.numpy as jnp
from jax import lax
from jax.experimental import pallas as pl
from jax.experimental.pallas import tpu as pltpu
from jax.experimental.pallas import tpu_sc as plsc


def _sc_embed(ids_flat, token_table, position_table, *, B, L, D):
    NC, NS = 2, 16
    NW = NC * NS
    n_rows = (B * L) // NW
    CH = 128
    NCH = n_rows // CH
    NBUF = 6

    mesh = plsc.VectorSubcoreMesh(core_axis_name="c", subcore_axis_name="s")

    @functools.partial(
        pl.kernel,
        mesh=mesh,
        out_type=jax.ShapeDtypeStruct((B * L, D), jnp.float32),
        scratch_types=[
            pltpu.VMEM((n_rows,), jnp.int32),
            pltpu.VMEM((L, D), jnp.float32),
        ] + [pltpu.VMEM((CH, D), jnp.float32) for _ in range(NBUF)]
          + [pltpu.SemaphoreType.DMA for _ in range(2 * NBUF + 2)],
    )
    def k(ids_hbm, tbl_hbm, pos_hbm, out_hbm, idx_v, pos_v, *rest):
        bufs = rest[:NBUF]
        gsem = rest[NBUF:2 * NBUF]
        ssem = rest[2 * NBUF:3 * NBUF]
        isem, psem = rest[3 * NBUF], rest[3 * NBUF + 1]

        wid = lax.axis_index("s") * NC + lax.axis_index("c")
        base = wid * n_rows
        idx_cp = pltpu.async_copy(ids_hbm.at[pl.ds(base, n_rows)], idx_v, isem)
        pos_cp = pltpu.async_copy(pos_hbm.at[pl.ds(0, L)], pos_v, psem)
        idx_cp.wait()

        def issue_gather(c):
            b = c % NBUF
            return pltpu.async_copy(
                tbl_hbm.at[idx_v.at[pl.ds(c * CH, CH)]], bufs[b], gsem[b])

        gathers = {c: issue_gather(c) for c in range(NBUF - 1)}
        scatters = {}
        pos_cp.wait()

        for c in range(NCH):
            b = c % NBUF
            gathers.pop(c).wait()

            buf, off = bufs[b], c * CH
            l0 = off % L
            n1 = min(CH, L - l0)

            def add_run(i0, cnt, lbase, buf=buf):
                def add_row(i, carry):
                    for j in range(D // 16):
                        sl = pl.ds(j * 16, 16)
                        buf[i0 + i, sl] = buf[i0 + i, sl] + pos_v[lbase + i, sl]
                    return carry
                lax.fori_loop(0, cnt, add_row, 0)

            add_run(0, n1, l0)
            if n1 < CH:
                add_run(n1, CH - n1, 0)

            scatters[c] = pltpu.async_copy(
                buf, out_hbm.at[pl.ds(base + off, CH)], ssem[b])

            if c + NBUF - 1 < NCH:
                if c >= 1:
                    scatters.pop(c - 1).wait()
                gathers[c + NBUF - 1] = issue_gather(c + NBUF - 1)

        for c in sorted(scatters):
            scatters[c].wait()

    return k(ids_flat, token_table, position_table)


def kernel(input_ids, token_table, position_table):
    B, L = input_ids.shape
    _, D = token_table.shape
    ids_flat = input_ids.reshape(B * L).astype(jnp.int32)
    out = _sc_embed(ids_flat, token_table.astype(jnp.float32),
                    position_table.astype(jnp.float32), B=B, L=L, D=D)
    return out.reshape(B, L, D)


# speedup vs baseline: 1.2358x; 1.0275x over previous
import functools

import jax
import jax.numpy as jnp
from jax import lax
from jax.experimental import pallas as pl
from jax.experimental.pallas import tpu as pltpu
from jax.experimental.pallas import tpu_sc as plsc


def _sc_embed(ids_flat, token_table, position_table, *, B, L, D):
    NC, NS = 2, 16
    NW = NC * NS
    BPW = B // NW
    n_rows = BPW * L
    NBUF = 3

    mesh = plsc.VectorSubcoreMesh(core_axis_name="c", subcore_axis_name="s")

    @functools.partial(
        pl.kernel,
        mesh=mesh,
        out_type=jax.ShapeDtypeStruct((B * L, D), jnp.float32),
        scratch_types=[
            pltpu.VMEM((n_rows,), jnp.int32),
            pltpu.VMEM((L, D), jnp.float32),
        ] + [pltpu.VMEM((L, D), jnp.float32) for _ in range(NBUF)]
          + [pltpu.SemaphoreType.DMA for _ in range(2 * NBUF + 2)],
    )
    def k(ids_hbm, tbl_hbm, pos_hbm, out_hbm, idx_v, pos_v, *rest):
        bufs = rest[:NBUF]
        gsem = rest[NBUF:2 * NBUF]
        ssem = rest[2 * NBUF:3 * NBUF]
        isem, psem = rest[3 * NBUF], rest[3 * NBUF + 1]

        wid = lax.axis_index("s") * NC + lax.axis_index("c")
        base = wid * n_rows
        idx_cp = pltpu.async_copy(ids_hbm.at[pl.ds(base, n_rows)], idx_v, isem)
        pos_cp = pltpu.async_copy(pos_hbm.at[pl.ds(0, L)], pos_v, psem)
        idx_cp.wait()

        def issue_gather(c):
            b = c % NBUF
            off = c * L
            cp1 = pltpu.async_copy(
                tbl_hbm.at[idx_v.at[pl.ds(off, 128)]],
                bufs[b].at[pl.ds(0, 128)], gsem[b])
            cp2 = pltpu.async_copy(
                tbl_hbm.at[idx_v.at[pl.ds(off + 128, L - 128)]],
                bufs[b].at[pl.ds(128, L - 128)], gsem[b])
            return (cp1, cp2)

        gathers = {0: issue_gather(0), 1: issue_gather(1)}
        scatters = {}
        pos_cp.wait()

        for c in range(BPW):
            b = c % NBUF
            cp1, cp2 = gathers.pop(c)
            cp1.wait()
            cp2.wait()

            if c + 2 < BPW:
                if c >= 1:
                    s1p, s2p = scatters.pop(c - 1)
                    s1p.wait()
                    s2p.wait()
                gathers[c + 2] = issue_gather(c + 2)

            buf = bufs[b]

            def add_row(l, carry, buf=buf):
                for j in range(D // 16):
                    sl = pl.ds(j * 16, 16)
                    buf[l, sl] = buf[l, sl] + pos_v[l, sl]
                return carry

            lax.fori_loop(0, 128, add_row, 0)
            s1 = pltpu.async_copy(
                buf.at[pl.ds(0, 128)],
                out_hbm.at[pl.ds(base + c * L, 128)], ssem[b])
            lax.fori_loop(128, L, add_row, 0)
            s2 = pltpu.async_copy(
                buf.at[pl.ds(128, L - 128)],
                out_hbm.at[pl.ds(base + c * L + 128, L - 128)], ssem[b])
            scatters[c] = (s1, s2)

        for c in sorted(scatters):
            scatters[c][0].wait()
            scatters[c][1].wait()

    return k(ids_flat, token_table, position_table)


def kernel(input_ids, token_table, position_table):
    B, L = input_ids.shape
    _, D = token_table.shape
    ids_flat = input_ids.reshape(B * L).astype(jnp.int32)
    out = _sc_embed(ids_flat, token_table.astype(jnp.float32),
                    position_table.astype(jnp.float32), B=B, L=L, D=D)
    return out.reshape(B, L, D)


# speedup vs baseline: 1.2377x; 1.0015x over previous
import functools

import jax
import jax.numpy as jnp
from jax import lax
from jax.experimental import pallas as pl
from jax.experimental.pallas import tpu as pltpu
from jax.experimental.pallas import tpu_sc as plsc


def _sc_embed(ids_flat, token_table, position_table, *, B, L, D):
    NC, NS = 2, 16
    NW = NC * NS
    BPW = B // NW
    n_rows = BPW * L
    NBUF = 3

    mesh = plsc.VectorSubcoreMesh(core_axis_name="c", subcore_axis_name="s")

    @functools.partial(
        pl.kernel,
        mesh=mesh,
        out_type=jax.ShapeDtypeStruct((B * L, D), jnp.float32),
        scratch_types=[
            pltpu.VMEM((n_rows,), jnp.int32),
            pltpu.VMEM((L, D), jnp.float32),
        ] + [pltpu.VMEM((L, D), jnp.float32) for _ in range(NBUF)]
          + [pltpu.SemaphoreType.DMA for _ in range(2 * NBUF + 2)],
    )
    def k(ids_hbm, tbl_hbm, pos_hbm, out_hbm, idx_v, pos_v, *rest):
        bufs = rest[:NBUF]
        gsem = rest[NBUF:2 * NBUF]
        ssem = rest[2 * NBUF:3 * NBUF]
        isem, psem = rest[3 * NBUF], rest[3 * NBUF + 1]

        wid = lax.axis_index("s") * NC + lax.axis_index("c")
        base = wid * n_rows
        idx_cp = pltpu.async_copy(ids_hbm.at[pl.ds(base, n_rows)], idx_v, isem)
        pos_cp = pltpu.async_copy(pos_hbm.at[pl.ds(0, L)], pos_v, psem)
        idx_cp.wait()

        def issue_gather(c):
            b = c % NBUF
            off = c * L
            cp1 = pltpu.async_copy(
                tbl_hbm.at[idx_v.at[pl.ds(off, 128)]],
                bufs[b].at[pl.ds(0, 128)], gsem[b])
            cp2 = pltpu.async_copy(
                tbl_hbm.at[idx_v.at[pl.ds(off + 128, L - 128)]],
                bufs[b].at[pl.ds(128, L - 128)], gsem[b])
            return (cp1, cp2)

        gathers = {0: issue_gather(0), 1: issue_gather(1)}
        scatters = {}
        pos_cp.wait()

        for c in range(BPW):
            b = c % NBUF
            cp1, cp2 = gathers.pop(c)
            cp1.wait()
            cp2.wait()

            buf = bufs[b]

            def add_row(l, carry, buf=buf):
                for j in range(D // 16):
                    sl = pl.ds(j * 16, 16)
                    buf[l, sl] = buf[l, sl] + pos_v[l, sl]
                return carry

            lax.fori_loop(0, L, add_row, 0)

            scatters[c] = pltpu.async_copy(
                buf, out_hbm.at[pl.ds(base + c * L, L)], ssem[b])

            if c + 2 < BPW:
                if c >= 1:
                    scatters.pop(c - 1).wait()
                gathers[c + 2] = issue_gather(c + 2)

        for c in sorted(scatters):
            scatters[c].wait()

    return k(ids_flat, token_table, position_table)


def kernel(input_ids, token_table, position_table):
    B, L = input_ids.shape
    _, D = token_table.shape
    ids_flat = input_ids.reshape(B * L).astype(jnp.int32)
    out = _sc_embed(ids_flat, token_table.astype(jnp.float32),
                    position_table.astype(jnp.float32), B=B, L=L, D=D)
    return out.reshape(B, L, D)
